# baseline (speedup 1.0000x reference)
import functools

import jax
import jax.numpy as jnp
from jax import lax
from jax.experimental import pallas as pl
from jax.experimental.pallas import tpu as pltpu

N = 100000
E = 3200000
HEADS = 4
NHID = 1
NCLASS = 40
NLAYERS = 4
DT = 1.0
ALPHA = 1.0
GAMMA = 1.0

_NEG = -1e30


def _leaky(v):
    return jnp.maximum(v, 0.2 * v)


def _layer(X, src_s, dstid, cs, cd, W0, bias_gat):
    Xext = jnp.concatenate([X, jnp.zeros((1,), X.dtype)])
    Xs = X[src_s]
    Xd = Xext[dstid]
    gmax = jax.ops.segment_max(Xs, dstid, num_segments=N + 1)[:N]
    gmin = jax.ops.segment_min(Xs, dstid, num_segments=N + 1)[:N]
    gmax = jnp.maximum(gmax, X)
    gmin = jnp.minimum(gmin, X)
    ext = jnp.where(cs[None, :] >= 0, gmax[:, None], gmin[:, None])
    m = _leaky(cs[None, :] * ext + cd[None, :] * X[:, None])

    pre = cs[None, :] * Xs[:, None] + cd[None, :] * Xd[:, None]
    l = _leaky(pre)
    mext = jnp.concatenate([m, jnp.zeros((1, HEADS), m.dtype)])
    e = jnp.exp(l - mext[dstid])
    s = jax.ops.segment_sum(e, dstid, num_segments=N + 1)[:N]
    w = jax.ops.segment_sum(e * Xs[:, None], dstid, num_segments=N + 1)[:N]
    lself = _leaky((cs + cd)[None, :] * X[:, None])
    eself = jnp.exp(lself - m)
    s = s + eself
    w = w + eself * X[:, None]
    return (w / s) * W0[None, :] + bias_gat[None, :]


def _readout_body(x_ref, w_ref, b_ref, o_ref):
    o_ref[:, :] = x_ref[:, :] * w_ref[:, :] + b_ref[:, :]


def _readout(X, W_read, b_read):
    BN = 800
    wr = W_read[:, 0][None, :]
    br = b_read[None, :]
    return pl.pallas_call(
        _readout_body,
        out_shape=jax.ShapeDtypeStruct((N, NCLASS), jnp.float32),
        grid=(N // BN,),
        in_specs=[
            pl.BlockSpec((BN, 1), lambda i: (i, 0)),
            pl.BlockSpec((1, NCLASS), lambda i: (0, 0)),
            pl.BlockSpec((1, NCLASS), lambda i: (0, 0)),
        ],
        out_specs=pl.BlockSpec((BN, NCLASS), lambda i: (i, 0)),
    )(X[:, None], wr, br)


def kernel(x, edge_index, W_gat, att_src, att_dst, bias_gat, W_read, b_read):
    src, dst = edge_index[0], edge_index[1]
    keys = jnp.where(src == dst, N, dst).astype(jnp.int32)
    order = jnp.argsort(keys)
    src_s = src[order]
    dstid = keys[order]

    W0 = W_gat[:, 0]
    cs = W0 * att_src[:, 0]
    cd = W0 * att_dst[:, 0]

    X = x
    Y = x
    X_list = [X]
    Y_list = [Y]
    for _ in range(NLAYERS):
        conv = _layer(X, src_s, dstid, cs, cd, W0, bias_gat)
        agg = jnp.mean(jax.nn.elu(conv), axis=1)
        Y = Y + DT * (agg - ALPHA * Y - GAMMA * X)
        X = X + DT * Y
        X_list.append(X)
        Y_list.append(Y)

    out = _readout(X, W_read, b_read)
    X_all = jnp.stack(X_list, axis=1)
    Y_all = jnp.stack(Y_list, axis=1)
    return (out, X_all, Y_all)


# speedup vs baseline: 5.0742x; 5.0742x over previous
import jax
import jax.numpy as jnp
from jax import lax
from jax.experimental import pallas as pl
from jax.experimental.pallas import tpu as pltpu
from jax.experimental.pallas import tpu_sc as plsc

N = 100000
E = 3200000
HEADS = 4
NCLASS = 40
NLAYERS = 4
DT = 1.0
ALPHA = 1.0
GAMMA = 1.0

NC = 2
NS = 16
NW = NC * NS
TPT = 3136
NP = NW * TPT
GPT = TPT // 16
TABW = GPT * 8 + 16
CW = 1024
CSH = 10
NBUF = 4
EPAD = E + NBUF * CW + 8
NEG = -1e30


def _splat(v):
    return jnp.repeat(v.astype(jnp.float32), 16)


def _layer_body(x_hbm, y_hbm, src_hbm, rplo_hbm, rphi_hbm, par_hbm,
                tab_hbm, xo_hbm, yo_hbm,
                xv, yv, rplo, rphi, win, xov, yov, pv, tab_v, sem):
    wid = lax.axis_index("s") * NC + lax.axis_index("c")
    tb = wid * TPT

    pltpu.sync_copy(x_hbm, xv)
    pltpu.sync_copy(y_hbm.at[pl.ds(tb, TPT)], yv)
    pltpu.sync_copy(rplo_hbm.at[pl.ds(tb, TPT)], rplo)
    pltpu.sync_copy(rphi_hbm.at[pl.ds(tb, TPT)], rphi)
    pltpu.sync_copy(tab_hbm.at[pl.ds(wid * TABW, TABW)], tab_v)
    pltpu.sync_copy(par_hbm, pv)

    cs = [pv[pl.ds(h * 16, 16)] for h in range(HEADS)]
    cd = [pv[pl.ds((HEADS + h) * 16, 16)] for h in range(HEADS)]
    w0 = [pv[pl.ds((2 * HEADS + h) * 16, 16)] for h in range(HEADS)]
    bg = [pv[pl.ds((3 * HEADS + h) * 16, 16)] for h in range(HEADS)]

    def group(g, _):
        nb = g * 16
        rp0 = rplo[pl.ds(nb, 16)]
        rp1 = rphi[pl.ds(nb, 16)]
        xd = xv[pl.ds(tb + nb, 16)]

        cdxd = [cd[h] * xd for h in range(HEADS)]
        rm, ss, ww = [], [], []
        for h in range(HEADS):
            pre = cs[h] * xd + cdxd[h]
            rm.append(jnp.maximum(pre, 0.2 * pre))
            ss.append(jnp.full((16,), 1.0, jnp.float32))
            ww.append(xd)

        tv = tab_v[pl.ds(g * 8, 16)]
        wa = tv[0]
        wend = tv[1]
        kmx = tv[2]
        ntrips = lax.shift_right_logical(wend - wa + (CW - 1), CSH)

        @plsc.parallel_loop(0, ntrips, carry=(*rm, *ss, *ww))
        def blk(i, st):
            rm = list(st[:HEADS])
            ss = list(st[HEADS:2 * HEADS])
            ww = list(st[2 * HEADS:3 * HEADS])
            slot = i & (NBUF - 1)
            sbase = slot * CW
            wb = pl.multiple_of(wa + i * CW, 8)
            pltpu.sync_copy(src_hbm.at[pl.ds(wb, CW)],
                            win.at[pl.ds(sbase, CW)])
            wbv = jnp.zeros((16,), jnp.int32) + wb
            lo = jnp.maximum(rp0 - wbv, 0)
            hi = jnp.minimum(rp1 - wbv, CW)
            cnt = hi - lo
            lob = lo + sbase

            @plsc.parallel_loop(0, kmx, carry=(*rm, *ss, *ww))
            def edges(j, st2):
                rm = list(st2[:HEADS])
                ss = list(st2[HEADS:2 * HEADS])
                ww = list(st2[2 * HEADS:3 * HEADS])
                msk = j < cnt
                idx = jnp.where(msk, lob + j, 0)
                sv = plsc.load_gather(win, [idx], mask=msk)
                sv = jnp.where(msk, sv, 0)
                xs = plsc.load_gather(xv, [sv], mask=msk)
                for h in range(HEADS):
                    pre = cs[h] * xs + cdxd[h]
                    l = jnp.maximum(pre, 0.2 * pre)
                    l = jnp.where(msk, l, NEG)
                    nm = jnp.maximum(rm[h], l)
                    scale = jnp.exp(rm[h] - nm)
                    p = jnp.exp(l - nm)
                    ss[h] = ss[h] * scale + p
                    ww[h] = ww[h] * scale + p * xs
                    rm[h] = nm
                return (*rm, *ss, *ww)

            return edges

        st = blk
        ss = list(st[HEADS:2 * HEADS])
        ww = list(st[2 * HEADS:3 * HEADS])

        agg = jnp.full((16,), 0.0, jnp.float32)
        for h in range(HEADS):
            outh = (ww[h] / ss[h]) * w0[h] + bg[h]
            eluh = jnp.where(outh > 0, outh, jnp.exp(outh) - 1.0)
            agg = agg + eluh
        agg = agg * (1.0 / HEADS)

        yd = yv[pl.ds(nb, 16)]
        yn = yd + DT * (agg - ALPHA * yd - GAMMA * xd)
        xn = xd + DT * yn
        xov[pl.ds(nb, 16)] = xn
        yov[pl.ds(nb, 16)] = yn
        return _

    lax.fori_loop(0, GPT, group, None)

    pltpu.sync_copy(xov, xo_hbm.at[pl.ds(tb, TPT)])
    pltpu.sync_copy(yov, yo_hbm.at[pl.ds(tb, TPT)])


_sc_layer = pl.kernel(
    _layer_body,
    out_type=(jax.ShapeDtypeStruct((NP,), jnp.float32),
              jax.ShapeDtypeStruct((NP,), jnp.float32)),
    mesh=plsc.VectorSubcoreMesh(core_axis_name="c", subcore_axis_name="s",
                                num_cores=NC, num_subcores=NS),
    compiler_params=pltpu.CompilerParams(needs_layout_passes=False),
    scratch_types=[
        pltpu.VMEM((NP,), jnp.float32),
        pltpu.VMEM((TPT,), jnp.float32),
        pltpu.VMEM((TPT,), jnp.int32),
        pltpu.VMEM((TPT,), jnp.int32),
        pltpu.VMEM((NBUF * CW,), jnp.int32),
        pltpu.VMEM((TPT,), jnp.float32),
        pltpu.VMEM((TPT,), jnp.float32),
        pltpu.VMEM((16 * HEADS * 4,), jnp.float32),
        pltpu.VMEM((TABW,), jnp.int32),
        pltpu.SemaphoreType.DMA,
    ],
)


def _readout_body(x_ref, w_ref, b_ref, o_ref):
    o_ref[:, :] = x_ref[:, :] * w_ref[:, :] + b_ref[:, :]


def _readout(X, W_read, b_read):
    BN = 800
    wr = W_read[:, 0][None, :]
    br = b_read[None, :]
    return pl.pallas_call(
        _readout_body,
        out_shape=jax.ShapeDtypeStruct((N, NCLASS), jnp.float32),
        grid=(N // BN,),
        in_specs=[
            pl.BlockSpec((BN, 1), lambda i: (i, 0)),
            pl.BlockSpec((1, NCLASS), lambda i: (0, 0)),
            pl.BlockSpec((1, NCLASS), lambda i: (0, 0)),
        ],
        out_specs=pl.BlockSpec((BN, NCLASS), lambda i: (i, 0)),
    )(X[:, None], wr, br)


def kernel(x, edge_index, W_gat, att_src, att_dst, bias_gat, W_read, b_read):
    src, dst = edge_index[0], edge_index[1]
    keys = jnp.where(src == dst, N, dst).astype(jnp.int32)
    order = jnp.argsort(keys)
    src_s = src[order].astype(jnp.int32)
    keys_s = keys[order]
    row_ptr = jnp.searchsorted(keys_s, jnp.arange(NP + 1, dtype=jnp.int32),
                               side="left").astype(jnp.int32)
    rp_lo = row_ptr[:NP]
    rp_hi = row_ptr[1:NP + 1]
    src_pad = jnp.concatenate([src_s, jnp.zeros((EPAD - E,), jnp.int32)])

    gnodes = (jnp.arange(NW) * TPT)[:, None] + 16 * jnp.arange(GPT + 1)[None, :]
    raw = row_ptr[gnodes]
    deg = (rp_hi - rp_lo).reshape(NW, GPT, 16)
    rec = jnp.stack([raw[:, :GPT] & ~7,
                     raw[:, 1:GPT + 1],
                     jnp.max(deg, axis=2),
                     ], axis=2)
    rec = jnp.pad(rec, ((0, 0), (0, 0), (0, 5))).reshape(NW, GPT * 8)
    tab = jnp.pad(rec, ((0, 0), (0, TABW - GPT * 8))).reshape(-1)

    W0 = W_gat[:, 0]
    cs = W0 * att_src[:, 0]
    cd = W0 * att_dst[:, 0]
    params = jnp.concatenate([_splat(cs), _splat(cd), _splat(W0),
                              _splat(bias_gat)])

    zpad = jnp.zeros((NP - N,), jnp.float32)
    X = jnp.concatenate([x, zpad])
    Y = X
    X_list = [X]
    Y_list = [Y]
    for _ in range(NLAYERS):
        X, Y = _sc_layer(X, Y, src_pad, rp_lo, rp_hi, params, tab)
        X_list.append(X)
        Y_list.append(Y)

    out = _readout(X[:N], W_read, b_read)
    X_all = jnp.stack([v[:N] for v in X_list], axis=1)
    Y_all = jnp.stack([v[:N] for v in Y_list], axis=1)
    return (out, X_all, Y_all)


# speedup vs baseline: 14.6740x; 2.8919x over previous
import jax
import jax.numpy as jnp
from jax import lax
from jax.experimental import pallas as pl
from jax.experimental.pallas import tpu as pltpu
from jax.experimental.pallas import tpu_sc as plsc

N = 100000
E = 3200000
HEADS = 4
NCLASS = 40
NLAYERS = 4
DT = 1.0
ALPHA = 1.0
GAMMA = 1.0

NC = 2
NS = 16
NW = NC * NS
TPT = 3136
NP = NW * TPT
GPT = TPT // 16
TABW = GPT * 8 + 16
CW = 1024
CSH = 10
NBUF = 4
EPAD = E + NBUF * CW + 8
NEG = -1e30


def _splat(v):
    return jnp.repeat(v.astype(jnp.float32), 16)


def _layer_body(x_hbm, y_hbm, src_hbm, rplo_hbm, rphi_hbm, par_hbm,
                tab_hbm, xo_hbm, yo_hbm,
                xv, yv, rplo, rphi, win, xov, yov, pv, tab_v, sem):
    wid = lax.axis_index("s") * NC + lax.axis_index("c")
    tb = wid * TPT

    pltpu.sync_copy(x_hbm, xv)
    pltpu.sync_copy(y_hbm.at[pl.ds(tb, TPT)], yv)
    pltpu.sync_copy(rplo_hbm.at[pl.ds(tb, TPT)], rplo)
    pltpu.sync_copy(rphi_hbm.at[pl.ds(tb, TPT)], rphi)
    pltpu.sync_copy(tab_hbm.at[pl.ds(wid * TABW, TABW)], tab_v)
    pltpu.sync_copy(par_hbm, pv)

    cs = [pv[pl.ds(h * 16, 16)] for h in range(HEADS)]
    cd = [pv[pl.ds((HEADS + h) * 16, 16)] for h in range(HEADS)]
    w0 = [pv[pl.ds((2 * HEADS + h) * 16, 16)] for h in range(HEADS)]
    bg = [pv[pl.ds((3 * HEADS + h) * 16, 16)] for h in range(HEADS)]

    def group(g, _):
        nb = g * 16
        rp0 = rplo[pl.ds(nb, 16)]
        rp1 = rphi[pl.ds(nb, 16)]
        xd = xv[pl.ds(tb + nb, 16)]

        cdxd = [cd[h] * xd for h in range(HEADS)]
        rm, ss, ww = [], [], []
        for h in range(HEADS):
            pre = cs[h] * xd + cdxd[h]
            rm.append(jnp.maximum(pre, 0.2 * pre))
            ss.append(jnp.full((16,), 1.0, jnp.float32))
            ww.append(xd)

        tv = tab_v[pl.ds(g * 8, 16)]
        wa = tv[0]
        wend = tv[1]
        kmx = tv[2]
        ntrips = lax.shift_right_logical(wend - wa + (CW - 1), CSH)

        @plsc.parallel_loop(0, ntrips, carry=(*rm, *ss, *ww))
        def blk(i, st):
            rm = list(st[:HEADS])
            ss = list(st[HEADS:2 * HEADS])
            ww = list(st[2 * HEADS:3 * HEADS])
            slot = i & (NBUF - 1)
            sbase = slot * CW
            wb = pl.multiple_of(wa + i * CW, 8)
            pltpu.sync_copy(src_hbm.at[pl.ds(wb, CW)],
                            win.at[pl.ds(sbase, CW)])
            wbv = jnp.zeros((16,), jnp.int32) + wb
            lo = jnp.maximum(rp0 - wbv, 0)
            hi = jnp.minimum(rp1 - wbv, CW)
            cnt = hi - lo
            lob = lo + sbase

            @plsc.parallel_loop(0, kmx, carry=(*rm, *ss, *ww))
            def edges(j, st2):
                rm = list(st2[:HEADS])
                ss = list(st2[HEADS:2 * HEADS])
                ww = list(st2[2 * HEADS:3 * HEADS])
                msk = j < cnt
                idx = jnp.where(msk, lob + j, 0)
                sv = plsc.load_gather(win, [idx], mask=msk)
                sv = jnp.where(msk, sv, 0)
                xs = plsc.load_gather(xv, [sv], mask=msk)
                for h in range(HEADS):
                    pre = cs[h] * xs + cdxd[h]
                    l = jnp.maximum(pre, 0.2 * pre)
                    l = jnp.where(msk, l, NEG)
                    nm = jnp.maximum(rm[h], l)
                    scale = jnp.exp(rm[h] - nm)
                    p = jnp.exp(l - nm)
                    ss[h] = ss[h] * scale + p
                    ww[h] = ww[h] * scale + p * xs
                    rm[h] = nm
                return (*rm, *ss, *ww)

            return edges

        st = blk
        ss = list(st[HEADS:2 * HEADS])
        ww = list(st[2 * HEADS:3 * HEADS])

        agg = jnp.full((16,), 0.0, jnp.float32)
        for h in range(HEADS):
            outh = (ww[h] / ss[h]) * w0[h] + bg[h]
            eluh = jnp.where(outh > 0, outh, jnp.exp(outh) - 1.0)
            agg = agg + eluh
        agg = agg * (1.0 / HEADS)

        yd = yv[pl.ds(nb, 16)]
        yn = yd + DT * (agg - ALPHA * yd - GAMMA * xd)
        xn = xd + DT * yn
        xov[pl.ds(nb, 16)] = xn
        yov[pl.ds(nb, 16)] = yn
        return _

    lax.fori_loop(0, GPT, group, None)

    pltpu.sync_copy(xov, xo_hbm.at[pl.ds(tb, TPT)])
    pltpu.sync_copy(yov, yo_hbm.at[pl.ds(tb, TPT)])


_sc_layer = pl.kernel(
    _layer_body,
    out_type=(jax.ShapeDtypeStruct((NP,), jnp.float32),
              jax.ShapeDtypeStruct((NP,), jnp.float32)),
    mesh=plsc.VectorSubcoreMesh(core_axis_name="c", subcore_axis_name="s",
                                num_cores=NC, num_subcores=NS),
    compiler_params=pltpu.CompilerParams(needs_layout_passes=False),
    scratch_types=[
        pltpu.VMEM((NP,), jnp.float32),
        pltpu.VMEM((TPT,), jnp.float32),
        pltpu.VMEM((TPT,), jnp.int32),
        pltpu.VMEM((TPT,), jnp.int32),
        pltpu.VMEM((NBUF * CW,), jnp.int32),
        pltpu.VMEM((TPT,), jnp.float32),
        pltpu.VMEM((TPT,), jnp.float32),
        pltpu.VMEM((16 * HEADS * 4,), jnp.float32),
        pltpu.VMEM((TABW,), jnp.int32),
        pltpu.SemaphoreType.DMA,
    ],
)


def _readout_body(x_ref, w_ref, b_ref, o_ref):
    o_ref[:, :] = x_ref[:, :] * w_ref[:, :] + b_ref[:, :]


def _readout(X, W_read, b_read):
    BN = 800
    wr = W_read[:, 0][None, :]
    br = b_read[None, :]
    return pl.pallas_call(
        _readout_body,
        out_shape=jax.ShapeDtypeStruct((N, NCLASS), jnp.float32),
        grid=(N // BN,),
        in_specs=[
            pl.BlockSpec((BN, 1), lambda i: (i, 0)),
            pl.BlockSpec((1, NCLASS), lambda i: (0, 0)),
            pl.BlockSpec((1, NCLASS), lambda i: (0, 0)),
        ],
        out_specs=pl.BlockSpec((BN, NCLASS), lambda i: (i, 0)),
    )(X[:, None], wr, br)


def kernel(x, edge_index, W_gat, att_src, att_dst, bias_gat, W_read, b_read):
    src, dst = edge_index[0], edge_index[1]
    keys = jnp.where(src == dst, N, dst).astype(jnp.int32)
    keys_s, src_s = lax.sort_key_val(keys, src.astype(jnp.int32))
    row_ptr = jnp.searchsorted(keys_s, jnp.arange(NP + 1, dtype=jnp.int32),
                               side="left", method="sort").astype(jnp.int32)
    rp_lo = row_ptr[:NP]
    rp_hi = row_ptr[1:NP + 1]
    src_pad = jnp.concatenate([src_s, jnp.zeros((EPAD - E,), jnp.int32)])

    gnodes = (jnp.arange(NW) * TPT)[:, None] + 16 * jnp.arange(GPT + 1)[None, :]
    raw = row_ptr[gnodes]
    deg = (rp_hi - rp_lo).reshape(NW, GPT, 16)
    rec = jnp.stack([raw[:, :GPT] & ~7,
                     raw[:, 1:GPT + 1],
                     jnp.max(deg, axis=2),
                     ], axis=2)
    rec = jnp.pad(rec, ((0, 0), (0, 0), (0, 5))).reshape(NW, GPT * 8)
    tab = jnp.pad(rec, ((0, 0), (0, TABW - GPT * 8))).reshape(-1)

    W0 = W_gat[:, 0]
    cs = W0 * att_src[:, 0]
    cd = W0 * att_dst[:, 0]
    params = jnp.concatenate([_splat(cs), _splat(cd), _splat(W0),
                              _splat(bias_gat)])

    zpad = jnp.zeros((NP - N,), jnp.float32)
    X = jnp.concatenate([x, zpad])
    Y = X
    X_list = [X]
    Y_list = [Y]
    for _ in range(NLAYERS):
        X, Y = _sc_layer(X, Y, src_pad, rp_lo, rp_hi, params, tab)
        X_list.append(X)
        Y_list.append(Y)

    out = _readout(X[:N], W_read, b_read)
    X_all = jnp.stack([v[:N] for v in X_list], axis=1)
    Y_all = jnp.stack([v[:N] for v in Y_list], axis=1)
    return (out, X_all, Y_all)
